# baseline (speedup 1.0000x reference)
import functools

import numpy as np
import jax
import jax.numpy as jnp
from jax import lax
from jax.experimental import pallas as pl
from jax.experimental.pallas import tpu as pltpu
from jax.experimental.pallas import tpu_sc as plsc

_N = 10000
_K = 8
_H = 128
_NH = 4
_DH = 32
_E = 8
_FH = 128

_BN = 200
_NB = _N // _BN
_BK = _BN * _K

_NW = 32
_CHUNK = 128
_NCH = 20
_BPAD = _NW * _CHUNK * _NCH

_PNAMES = (
    'ln_attn_g', 'ln_attn_b', 'We', 'be', 'Wn', 'bn', 'Wm', 'bm',
    'Wq', 'bq', 'Wk', 'bk', 'Wv', 'bv', 'Wo', 'bo',
    'ln_ffn_ng', 'ln_ffn_nb', 'ln_ffn_eg', 'ln_ffn_eb',
    'Wn1', 'bn1', 'Wn2', 'bn2', 'We1', 'be1', 'We2', 'be2',
)


def _build_selectors():
  selk = np.zeros((_K, _H, _NH * _K), np.float32)
  expk = np.zeros((_K, _NH * _K, _H), np.float32)
  for k in range(_K):
    for h in range(_NH):
      selk[k, h * _DH:(h + 1) * _DH, h * _K + k] = 1.0
      expk[k, h * _K + k, h * _DH:(h + 1) * _DH] = 1.0
  msel = np.zeros((_K, _NH * _K), np.float32)
  onesb = np.zeros((_NH * _K, _NH * _K), np.float32)
  for h in range(_NH):
    for k in range(_K):
      msel[k, h * _K + k] = 1.0
      onesb[h * _K + k, h * _K:(h + 1) * _K] = 1.0
  return selk, expk, msel, onesb


_SELK, _EXPK, _MSEL, _ONESB = _build_selectors()


def _sc_gather(table, idx3):
  mesh = plsc.VectorSubcoreMesh(core_axis_name="c", subcore_axis_name="s")

  @functools.partial(
      pl.kernel,
      mesh=mesh,
      out_type=jax.ShapeDtypeStruct((_BPAD, _H), jnp.float32),
      scratch_types=[
          pltpu.VMEM((_NCH, _CHUNK), jnp.int32),
          pltpu.VMEM((2, _CHUNK, _H), jnp.float32),
          pltpu.SemaphoreType.DMA,
          pltpu.SemaphoreType.DMA,
      ],
  )
  def gk(table_hbm, idx_hbm, out_hbm, idx_v, rows_v, sem0, sem1):
    c = lax.axis_index("c")
    s = lax.axis_index("s")
    wid = s * 2 + c
    base = wid * (_NCH * _CHUNK)
    pltpu.sync_copy(idx_hbm.at[wid], idx_v)
    sems = (sem0, sem1)
    handles = [None, None]
    for j in range(_NCH):
      b = j % 2
      handles[b] = pltpu.async_copy(table_hbm.at[idx_v.at[j]], rows_v.at[b],
                                    sems[b])
      if j > 0:
        pb = (j - 1) % 2
        handles[pb].wait()
        pltpu.sync_copy(rows_v.at[pb],
                        out_hbm.at[pl.ds(base + (j - 1) * _CHUNK, _CHUNK)])
    lb = (_NCH - 1) % 2
    handles[lb].wait()
    pltpu.sync_copy(rows_v.at[lb],
                    out_hbm.at[pl.ds(base + (_NCH - 1) * _CHUNK, _CHUNK)])

  return gk(table, idx3)


def _rep8(x):
  r, c = x.shape
  return jnp.broadcast_to(x[:, None, :], (r, 8, c)).reshape(r * 8, c)


def _silu(x):
  return x / (1.0 + jnp.exp(-x))


def _lncore(x):
  m = jnp.mean(x, -1, keepdims=True)
  xc = x - m
  v = jnp.mean(xc * xc, -1, keepdims=True)
  return xc * lax.rsqrt(v + 1e-5)


def _dot(a, b):
  return jnp.dot(a, b, preferred_element_type=jnp.float32)


def _dotb(a, b16):
  return jnp.dot(a.astype(jnp.bfloat16), b16,
                 preferred_element_type=jnp.float32)


def _tc_body(nf_ref, nbr_ref, ea_ref, ef_ref, nm_ref, am_ref, wg_ref,
             selk_ref, expk_ref, msel_ref, onesb_ref,
             lag_ref, lab_ref, We_ref, be_ref, Wn_ref, bn_ref, Wm_ref, bm_ref,
             Wq_ref, bq_ref, Wk_ref, bk_ref, Wv_ref, bv_ref, Wo_ref, bo_ref,
             lng_ref, lnb_ref, leg_ref, leb_ref,
             Wn1_ref, bn1_ref, Wn2_ref, bn2_ref, We1_ref, be1_ref, We2_ref,
             be2_ref, no_ref, eo_ref, hcc_ref, hnc_ref, gat_ref):
  e = pl.program_id(1)
  nf = nf_ref[...]
  ea = ea_ref[...]
  ef = ef_ref[...]
  nmask = nm_ref[...]
  am2 = am_ref[...].reshape(_BK, _K)

  @pl.when(e == 0)
  def _prologue():
    hcc_ref[...] = _lncore(nf)
    hnc_ref[...] = _lncore(nbr_ref[...])
    logits = _dot(nf, wg_ref[...])
    iot = lax.broadcasted_iota(jnp.int32, (_BN, _E), 1)
    m1 = jnp.max(logits, -1, keepdims=True)
    i1 = jnp.min(jnp.where(logits == m1, iot, _E), -1, keepdims=True)
    rest = jnp.where(iot == i1, -1e30, logits)
    m2 = jnp.max(rest, -1, keepdims=True)
    i2 = jnp.min(jnp.where(rest == m2, iot, _E), -1, keepdims=True)
    e21 = jnp.exp(m2 - m1)
    p1 = 1.0 / (1.0 + e21)
    p2 = 1.0 - p1
    gat_ref[...] = (jnp.where(iot == i1, p1, 0.0) +
                    jnp.where(iot == i2, p2, 0.0))
    no_ref[...] = jnp.zeros((_BN, _H), jnp.float32)
    eo_ref[...] = jnp.zeros((_BK, _H), jnp.float32)

  hc_core = hcc_ref[...]
  hn_core = hnc_ref[...]
  gates = gat_ref[...]

  amS = _dot(am2, msel_ref[...])
  nmb = jnp.broadcast_to(nmask[:, :, None], (_BN, _K, _H)).reshape(_BK, _H)
  cnt = jnp.sum(nmask, -1, keepdims=True) + 1e-5
  onesb = onesb_ref[...]

  ga = lag_ref[0]
  ba = lab_ref[0]
  hc = hc_core * ga + ba
  hn = hn_core * ga + ba

  edge_hidden = _silu(_dot(ea, We_ref[0]) + be_ref[0])
  Wn_e = Wn_ref[0]
  cm = _dotb(hc, Wn_e[:_H])
  node_hidden = _silu(_rep8(cm) + _dotb(hn, Wn_e[_H:]) + bn_ref[0])
  Wm_e = Wm_ref[0]
  message = _silu(_dotb(edge_hidden, Wm_e[:_H]) +
                  _dotb(node_hidden, Wm_e[_H:]) + bm_ref[0])
  mb = message.astype(jnp.bfloat16)

  scale = np.float32(1.0 / np.sqrt(_DH))
  q16 = ((jnp.dot(mb, Wq_ref[0], preferred_element_type=jnp.float32) +
          bq_ref[0]) * scale).astype(jnp.bfloat16)
  kk16 = (jnp.dot(mb, Wk_ref[0], preferred_element_type=jnp.float32) +
          bk_ref[0]).astype(jnp.bfloat16)
  v = jnp.dot(mb, Wv_ref[0], preferred_element_type=jnp.float32) + bv_ref[0]
  kk3 = kk16.reshape(_BN, _K, _H)
  v3 = v.reshape(_BN, _K, _H)

  S = amS
  for k in range(_K):
    S = S + jnp.dot(q16 * _rep8(kk3[:, k, :]), selk_ref[k],
                    preferred_element_type=jnp.float32)
  Smax = jnp.max(S, -1, keepdims=True)
  Ex = jnp.exp(S - Smax)
  A16 = (Ex / _dot(Ex, onesb)).astype(jnp.bfloat16)

  o = jnp.zeros((_BK, _H), jnp.float32)
  for k in range(_K):
    o = o + jnp.dot(A16, expk_ref[k],
                    preferred_element_type=jnp.float32) * _rep8(v3[:, k, :])
  edge_out = _dotb(o, Wo_ref[0]) + bo_ref[0]

  em = (edge_out * nmb).reshape(_BN, _K, _H)
  node_sum = em[:, 0, :]
  for k in range(1, _K):
    node_sum = node_sum + em[:, k, :]
  node_out = node_sum / cnt

  node_f = node_out + nf
  edge_f = edge_out + ef
  nh = _lncore(node_f) * lng_ref[0] + lnb_ref[0]
  nh = _dotb(_silu(_dotb(nh, Wn1_ref[0]) + bn1_ref[0]),
             Wn2_ref[0]) + bn2_ref[0]
  eh = _lncore(edge_f) * leg_ref[0] + leb_ref[0]
  eh = _dotb(_silu(_dotb(eh, We1_ref[0]) + be1_ref[0]),
             We2_ref[0]) + be2_ref[0]

  iot = lax.broadcasted_iota(jnp.int32, (_BN, _E), 1)
  ge = jnp.sum(jnp.where(iot == e, gates, 0.0), -1, keepdims=True)
  geb = jnp.broadcast_to(ge, (_BN, _H))
  no_ref[...] += geb * (node_f + nh)
  eo_ref[...] += _rep8(geb) * (edge_f + eh)


def _run_main(params, nf, nbr_flat, ea_flat, ef_flat, nmask, amask):
  full = lambda a: pl.BlockSpec(a.shape, lambda i: (0,) * a.ndim)
  selk = jnp.asarray(_SELK, jnp.bfloat16)
  expk = jnp.asarray(_EXPK, jnp.bfloat16)
  msel = jnp.asarray(_MSEL)
  onesb = jnp.asarray(_ONESB)
  bf = ('We', 'Wn', 'Wm', 'Wq', 'Wk', 'Wv', 'Wo', 'Wn1', 'Wn2', 'We1', 'We2')
  pvals = [params[n].astype(jnp.bfloat16) if n in bf else
           params[n].reshape(_E, 1, -1) for n in _PNAMES]
  ea_flat = ea_flat.astype(jnp.bfloat16)

  full = lambda a: pl.BlockSpec(a.shape, lambda i, j: (0,) * a.ndim)
  def perexp(a):
    return pl.BlockSpec((1,) + a.shape[1:],
                        lambda i, j: (j,) + (0,) * (a.ndim - 1))

  in_specs = [
      pl.BlockSpec((_BN, _H), lambda i, j: (i, 0)),
      pl.BlockSpec((_BK, _H), lambda i, j: (i, 0)),
      pl.BlockSpec((_BK, _H), lambda i, j: (i, 0)),
      pl.BlockSpec((_BK, _H), lambda i, j: (i, 0)),
      pl.BlockSpec((_BN, _K), lambda i, j: (i, 0)),
      pl.BlockSpec((_BN, _K, _K), lambda i, j: (i, 0, 0)),
      full(params['w_gate']),
      full(selk), full(expk), full(msel), full(onesb),
  ] + [perexp(p) for p in pvals]

  out_specs = [
      pl.BlockSpec((_BN, _H), lambda i, j: (i, 0)),
      pl.BlockSpec((_BK, _H), lambda i, j: (i, 0)),
  ]
  out_shape = [
      jax.ShapeDtypeStruct((_N, _H), jnp.float32),
      jax.ShapeDtypeStruct((_N * _K, _H), jnp.float32),
  ]
  return pl.pallas_call(
      _tc_body,
      grid=(_NB, _E),
      in_specs=in_specs,
      out_specs=out_specs,
      out_shape=out_shape,
      scratch_shapes=[
          pltpu.VMEM((_BN, _H), jnp.float32),
          pltpu.VMEM((_BK, _H), jnp.float32),
          pltpu.VMEM((_BN, _E), jnp.float32),
      ],
  )(nf, nbr_flat, ea_flat, ef_flat, nmask, amask, params['w_gate'],
    selk, expk, msel, onesb, *pvals)


def kernel(params, node_features, edge_features, edge_attr, neighbor_mask,
           attn_mask, neighbor_list):
  nf = node_features.astype(jnp.float32)
  idxf = neighbor_list.reshape(-1).astype(jnp.int32)
  idxp = jnp.concatenate(
      [idxf, jnp.zeros((_BPAD - _N * _K,), jnp.int32)])
  nbr_raw = _sc_gather(nf, idxp.reshape(_NW, _NCH, _CHUNK))
  nbr_flat = nbr_raw[:_N * _K]
  node_out, edge_flat = _run_main(
      params, nf, nbr_flat,
      edge_attr.reshape(_N * _K, _H).astype(jnp.float32),
      edge_features.reshape(_N * _K, _H).astype(jnp.float32),
      neighbor_mask.astype(jnp.float32), attn_mask.astype(jnp.float32))
  return node_out, edge_flat.reshape(_N, _K, _H)


# speedup vs baseline: 1.0031x; 1.0031x over previous
import functools

import numpy as np
import jax
import jax.numpy as jnp
from jax import lax
from jax.experimental import pallas as pl
from jax.experimental.pallas import tpu as pltpu
from jax.experimental.pallas import tpu_sc as plsc

_N = 10000
_K = 8
_H = 128
_NH = 4
_DH = 32
_E = 8
_FH = 128

_BN = 400
_NB = _N // _BN
_BK = _BN * _K

_NW = 32
_CHUNK = 128
_NCH = 20
_BPAD = _NW * _CHUNK * _NCH

_PNAMES = (
    'ln_attn_g', 'ln_attn_b', 'We', 'be', 'Wn', 'bn', 'Wm', 'bm',
    'Wq', 'bq', 'Wk', 'bk', 'Wv', 'bv', 'Wo', 'bo',
    'ln_ffn_ng', 'ln_ffn_nb', 'ln_ffn_eg', 'ln_ffn_eb',
    'Wn1', 'bn1', 'Wn2', 'bn2', 'We1', 'be1', 'We2', 'be2',
)


def _build_selectors():
  selk = np.zeros((_K, _H, _NH * _K), np.float32)
  expk = np.zeros((_K, _NH * _K, _H), np.float32)
  for k in range(_K):
    for h in range(_NH):
      selk[k, h * _DH:(h + 1) * _DH, h * _K + k] = 1.0
      expk[k, h * _K + k, h * _DH:(h + 1) * _DH] = 1.0
  msel = np.zeros((_K, _NH * _K), np.float32)
  onesb = np.zeros((_NH * _K, _NH * _K), np.float32)
  for h in range(_NH):
    for k in range(_K):
      msel[k, h * _K + k] = 1.0
      onesb[h * _K + k, h * _K:(h + 1) * _K] = 1.0
  return selk, expk, msel, onesb


_SELK, _EXPK, _MSEL, _ONESB = _build_selectors()


def _sc_gather(table, idx3):
  mesh = plsc.VectorSubcoreMesh(core_axis_name="c", subcore_axis_name="s")

  @functools.partial(
      pl.kernel,
      mesh=mesh,
      out_type=jax.ShapeDtypeStruct((_BPAD, _H), jnp.float32),
      scratch_types=[
          pltpu.VMEM((_NCH, _CHUNK), jnp.int32),
          pltpu.VMEM((2, _CHUNK, _H), jnp.float32),
          pltpu.SemaphoreType.DMA,
          pltpu.SemaphoreType.DMA,
      ],
  )
  def gk(table_hbm, idx_hbm, out_hbm, idx_v, rows_v, sem0, sem1):
    c = lax.axis_index("c")
    s = lax.axis_index("s")
    wid = s * 2 + c
    base = wid * (_NCH * _CHUNK)
    pltpu.sync_copy(idx_hbm.at[wid], idx_v)
    sems = (sem0, sem1)
    handles = [None, None]
    for j in range(_NCH):
      b = j % 2
      handles[b] = pltpu.async_copy(table_hbm.at[idx_v.at[j]], rows_v.at[b],
                                    sems[b])
      if j > 0:
        pb = (j - 1) % 2
        handles[pb].wait()
        pltpu.sync_copy(rows_v.at[pb],
                        out_hbm.at[pl.ds(base + (j - 1) * _CHUNK, _CHUNK)])
    lb = (_NCH - 1) % 2
    handles[lb].wait()
    pltpu.sync_copy(rows_v.at[lb],
                    out_hbm.at[pl.ds(base + (_NCH - 1) * _CHUNK, _CHUNK)])

  return gk(table, idx3)


def _rep8(x):
  r, c = x.shape
  return jnp.broadcast_to(x[:, None, :], (r, 8, c)).reshape(r * 8, c)


def _silu(x):
  return x / (1.0 + jnp.exp(-x))


def _lncore(x):
  m = jnp.mean(x, -1, keepdims=True)
  xc = x - m
  v = jnp.mean(xc * xc, -1, keepdims=True)
  return xc * lax.rsqrt(v + 1e-5)


def _dot(a, b):
  return jnp.dot(a, b, preferred_element_type=jnp.float32)


def _dotb(a, b16):
  return jnp.dot(a.astype(jnp.bfloat16), b16,
                 preferred_element_type=jnp.float32)


def _tc_body(nf_ref, nbr_ref, ea_ref, ef_ref, nm_ref, am_ref, wg_ref,
             selk_ref, expk_ref, msel_ref, onesb_ref,
             lag_ref, lab_ref, We_ref, be_ref, Wn_ref, bn_ref, Wm_ref, bm_ref,
             Wq_ref, bq_ref, Wk_ref, bk_ref, Wv_ref, bv_ref, Wo_ref, bo_ref,
             lng_ref, lnb_ref, leg_ref, leb_ref,
             Wn1_ref, bn1_ref, Wn2_ref, bn2_ref, We1_ref, be1_ref, We2_ref,
             be2_ref, no_ref, eo_ref, hcc_ref, hnc_ref, gat_ref, ams_ref,
             w_ref):
  e = pl.program_id(1)
  nf = nf_ref[...]
  ea = ea_ref[...]
  ef = ef_ref[...]

  @pl.when(e == 0)
  def _prologue():
    nmask = nm_ref[...]
    am2 = am_ref[...].reshape(_BK, _K)
    ams_ref[...] = _dot(am2, msel_ref[...])
    w_ref[...] = nmask / (jnp.sum(nmask, -1, keepdims=True) + 1e-5)
    hcc_ref[...] = _lncore(nf)
    hnc_ref[...] = _lncore(nbr_ref[...])
    logits = _dot(nf, wg_ref[...])
    iot = lax.broadcasted_iota(jnp.int32, (_BN, _E), 1)
    m1 = jnp.max(logits, -1, keepdims=True)
    i1 = jnp.min(jnp.where(logits == m1, iot, _E), -1, keepdims=True)
    rest = jnp.where(iot == i1, -1e30, logits)
    m2 = jnp.max(rest, -1, keepdims=True)
    i2 = jnp.min(jnp.where(rest == m2, iot, _E), -1, keepdims=True)
    e21 = jnp.exp(m2 - m1)
    p1 = 1.0 / (1.0 + e21)
    p2 = 1.0 - p1
    gat_ref[...] = (jnp.where(iot == i1, p1, 0.0) +
                    jnp.where(iot == i2, p2, 0.0))
    no_ref[...] = jnp.zeros((_BN, _H), jnp.float32)
    eo_ref[...] = jnp.zeros((_BK, _H), jnp.float32)

  hc_core = hcc_ref[...]
  hn_core = hnc_ref[...]
  gates = gat_ref[...]
  amS = ams_ref[...]
  w = w_ref[...]
  onesb = onesb_ref[...]

  ga = lag_ref[0]
  ba = lab_ref[0]
  hc = hc_core * ga + ba
  hn = hn_core * ga + ba

  edge_hidden = _silu(_dot(ea, We_ref[0]) + be_ref[0])
  Wn_e = Wn_ref[0]
  cm = _dotb(hc, Wn_e[:_H])
  node_hidden = _silu(_rep8(cm) + _dotb(hn, Wn_e[_H:]) + bn_ref[0])
  Wm_e = Wm_ref[0]
  message = _silu(_dotb(edge_hidden, Wm_e[:_H]) +
                  _dotb(node_hidden, Wm_e[_H:]) + bm_ref[0])
  mb = message.astype(jnp.bfloat16)

  scale = np.float32(1.0 / np.sqrt(_DH))
  q16 = ((jnp.dot(mb, Wq_ref[0], preferred_element_type=jnp.float32) +
          bq_ref[0]) * scale).astype(jnp.bfloat16)
  kk16 = (jnp.dot(mb, Wk_ref[0], preferred_element_type=jnp.float32) +
          bk_ref[0]).astype(jnp.bfloat16)
  v = jnp.dot(mb, Wv_ref[0], preferred_element_type=jnp.float32) + bv_ref[0]
  kk3 = kk16.reshape(_BN, _K, _H)
  v3 = v.reshape(_BN, _K, _H)

  S = amS
  for k in range(_K):
    S = S + jnp.dot(q16 * _rep8(kk3[:, k, :]), selk_ref[k],
                    preferred_element_type=jnp.float32)
  Smax = jnp.max(S, -1, keepdims=True)
  Ex = jnp.exp(S - Smax)
  A16 = (Ex / _dot(Ex, onesb)).astype(jnp.bfloat16)

  o = jnp.zeros((_BK, _H), jnp.float32)
  for k in range(_K):
    o = o + jnp.dot(A16, expk_ref[k],
                    preferred_element_type=jnp.float32) * _rep8(v3[:, k, :])
  edge_out = _dotb(o, Wo_ref[0]) + bo_ref[0]

  em = edge_out.reshape(_BN, _K, _H)
  node_out = em[:, 0, :] * w[:, 0:1]
  for k in range(1, _K):
    node_out = node_out + em[:, k, :] * w[:, k:k + 1]

  node_f = node_out + nf
  edge_f = edge_out + ef
  nh = _lncore(node_f) * lng_ref[0] + lnb_ref[0]
  nh = _dotb(_silu(_dotb(nh, Wn1_ref[0]) + bn1_ref[0]),
             Wn2_ref[0]) + bn2_ref[0]
  eh = _lncore(edge_f) * leg_ref[0] + leb_ref[0]
  eh = _dotb(_silu(_dotb(eh, We1_ref[0]) + be1_ref[0]),
             We2_ref[0]) + be2_ref[0]

  iot = lax.broadcasted_iota(jnp.int32, (_BN, _E), 1)
  ge = jnp.sum(jnp.where(iot == e, gates, 0.0), -1, keepdims=True)
  geb = jnp.broadcast_to(ge, (_BN, _H))
  no_ref[...] += geb * (node_f + nh)
  eo_ref[...] += _rep8(geb) * (edge_f + eh)


def _run_main(params, nf, nbr_flat, ea_flat, ef_flat, nmask, amask):
  full = lambda a: pl.BlockSpec(a.shape, lambda i: (0,) * a.ndim)
  selk = jnp.asarray(_SELK, jnp.bfloat16)
  expk = jnp.asarray(_EXPK, jnp.bfloat16)
  msel = jnp.asarray(_MSEL)
  onesb = jnp.asarray(_ONESB)
  bf = ('We', 'Wn', 'Wm', 'Wq', 'Wk', 'Wv', 'Wo', 'Wn1', 'Wn2', 'We1', 'We2')
  pvals = [params[n].astype(jnp.bfloat16) if n in bf else
           params[n].reshape(_E, 1, -1) for n in _PNAMES]
  ea_flat = ea_flat.astype(jnp.bfloat16)

  full = lambda a: pl.BlockSpec(a.shape, lambda i, j: (0,) * a.ndim)
  def perexp(a):
    return pl.BlockSpec((1,) + a.shape[1:],
                        lambda i, j: (j,) + (0,) * (a.ndim - 1))

  in_specs = [
      pl.BlockSpec((_BN, _H), lambda i, j: (i, 0)),
      pl.BlockSpec((_BK, _H), lambda i, j: (i, 0)),
      pl.BlockSpec((_BK, _H), lambda i, j: (i, 0)),
      pl.BlockSpec((_BK, _H), lambda i, j: (i, 0)),
      pl.BlockSpec((_BN, _K), lambda i, j: (i, 0)),
      pl.BlockSpec((_BN, _K, _K), lambda i, j: (i, 0, 0)),
      full(params['w_gate']),
      full(selk), full(expk), full(msel), full(onesb),
  ] + [perexp(p) for p in pvals]

  out_specs = [
      pl.BlockSpec((_BN, _H), lambda i, j: (i, 0)),
      pl.BlockSpec((_BK, _H), lambda i, j: (i, 0)),
  ]
  out_shape = [
      jax.ShapeDtypeStruct((_N, _H), jnp.float32),
      jax.ShapeDtypeStruct((_N * _K, _H), jnp.float32),
  ]
  return pl.pallas_call(
      _tc_body,
      grid=(_NB, _E),
      in_specs=in_specs,
      out_specs=out_specs,
      out_shape=out_shape,
      scratch_shapes=[
          pltpu.VMEM((_BN, _H), jnp.float32),
          pltpu.VMEM((_BK, _H), jnp.float32),
          pltpu.VMEM((_BN, _E), jnp.float32),
          pltpu.VMEM((_BK, _NH * _K), jnp.float32),
          pltpu.VMEM((_BN, _K), jnp.float32),
      ],
  )(nf, nbr_flat, ea_flat, ef_flat, nmask, amask, params['w_gate'],
    selk, expk, msel, onesb, *pvals)


def kernel(params, node_features, edge_features, edge_attr, neighbor_mask,
           attn_mask, neighbor_list):
  nf = node_features.astype(jnp.float32)
  idxf = neighbor_list.reshape(-1).astype(jnp.int32)
  idxp = jnp.concatenate(
      [idxf, jnp.zeros((_BPAD - _N * _K,), jnp.int32)])
  nbr_raw = _sc_gather(nf, idxp.reshape(_NW, _NCH, _CHUNK))
  nbr_flat = nbr_raw[:_N * _K]
  node_out, edge_flat = _run_main(
      params, nf, nbr_flat,
      edge_attr.reshape(_N * _K, _H).astype(jnp.float32),
      edge_features.reshape(_N * _K, _H).astype(jnp.float32),
      neighbor_mask.astype(jnp.float32), attn_mask.astype(jnp.float32))
  return node_out, edge_flat.reshape(_N, _K, _H)


# speedup vs baseline: 1.0329x; 1.0297x over previous
import functools

import numpy as np
import jax
import jax.numpy as jnp
from jax import lax
from jax.experimental import pallas as pl
from jax.experimental.pallas import tpu as pltpu
from jax.experimental.pallas import tpu_sc as plsc

_N = 10000
_K = 8
_H = 128
_NH = 4
_DH = 32
_E = 8
_FH = 128

_BN = 400
_NB = _N // _BN
_BK = _BN * _K

_NW = 32
_CHUNK = 128
_NCH = 20
_BPAD = _NW * _CHUNK * _NCH

_PNAMES = (
    'ln_attn_g', 'ln_attn_b', 'We', 'be', 'Wn', 'bn', 'Wm', 'bm',
    'Wq', 'bq', 'Wk', 'bk', 'Wv', 'bv', 'Wo', 'bo',
    'ln_ffn_ng', 'ln_ffn_nb', 'ln_ffn_eg', 'ln_ffn_eb',
    'Wn1', 'bn1', 'Wn2', 'bn2', 'We1', 'be1', 'We2', 'be2',
)


def _build_selectors():
  selk = np.zeros((_K, _H, _NH * _K), np.float32)
  expk = np.zeros((_K, _NH * _K, _H), np.float32)
  for k in range(_K):
    for h in range(_NH):
      selk[k, h * _DH:(h + 1) * _DH, h * _K + k] = 1.0
      expk[k, h * _K + k, h * _DH:(h + 1) * _DH] = 1.0
  msel = np.zeros((_K, _NH * _K), np.float32)
  onesb = np.zeros((_NH * _K, _NH * _K), np.float32)
  for h in range(_NH):
    for k in range(_K):
      msel[k, h * _K + k] = 1.0
      onesb[h * _K + k, h * _K:(h + 1) * _K] = 1.0
  return selk, expk, msel, onesb


_SELK, _EXPK, _MSEL, _ONESB = _build_selectors()


def _sc_gather(table, idx3):
  mesh = plsc.VectorSubcoreMesh(core_axis_name="c", subcore_axis_name="s")

  @functools.partial(
      pl.kernel,
      mesh=mesh,
      out_type=jax.ShapeDtypeStruct((_BPAD, _H), jnp.float32),
      scratch_types=[
          pltpu.VMEM((_NCH, _CHUNK), jnp.int32),
          pltpu.VMEM((2, _CHUNK, _H), jnp.float32),
          pltpu.SemaphoreType.DMA,
          pltpu.SemaphoreType.DMA,
      ],
  )
  def gk(table_hbm, idx_hbm, out_hbm, idx_v, rows_v, sem0, sem1):
    c = lax.axis_index("c")
    s = lax.axis_index("s")
    wid = s * 2 + c
    base = wid * (_NCH * _CHUNK)
    pltpu.sync_copy(idx_hbm.at[wid], idx_v)
    sems = (sem0, sem1)
    handles = [None, None]
    for j in range(_NCH):
      b = j % 2
      handles[b] = pltpu.async_copy(table_hbm.at[idx_v.at[j]], rows_v.at[b],
                                    sems[b])
      if j > 0:
        pb = (j - 1) % 2
        handles[pb].wait()
        pltpu.sync_copy(rows_v.at[pb],
                        out_hbm.at[pl.ds(base + (j - 1) * _CHUNK, _CHUNK)])
    lb = (_NCH - 1) % 2
    handles[lb].wait()
    pltpu.sync_copy(rows_v.at[lb],
                    out_hbm.at[pl.ds(base + (_NCH - 1) * _CHUNK, _CHUNK)])

  return gk(table, idx3)


def _rep8(x):
  r, c = x.shape
  return jnp.broadcast_to(x[:, None, :], (r, 8, c)).reshape(r * 8, c)


def _silu(x):
  return x / (1.0 + jnp.exp(-x))


def _lncore(x):
  m = jnp.mean(x, -1, keepdims=True)
  xc = x - m
  v = jnp.mean(xc * xc, -1, keepdims=True)
  return xc * lax.rsqrt(v + 1e-5)


def _dot(a, b):
  return jnp.dot(a, b, preferred_element_type=jnp.float32)


def _dotb(a, b16):
  return jnp.dot(a.astype(jnp.bfloat16), b16,
                 preferred_element_type=jnp.float32)


def _tc_body(nf_ref, nbr_ref, ea_ref, ef_ref, nm_ref, am_ref, wg_ref,
             selk_ref, expk_ref, msel_ref, onesb_ref,
             lag_ref, lab_ref, We_ref, be_ref, Wn_ref, bn_ref, Wm_ref, bm_ref,
             Wq_ref, bq_ref, Wk_ref, bk_ref, Wv_ref, bv_ref, Wo_ref, bo_ref,
             lng_ref, lnb_ref, leg_ref, leb_ref,
             Wn1_ref, bn1_ref, Wn2_ref, bn2_ref, We1_ref, be1_ref, We2_ref,
             be2_ref, no_ref, eo_ref, hcc_ref, hnc_ref, gat_ref):
  e = pl.program_id(1)
  nf = nf_ref[...]
  ea = ea_ref[...]
  ef = ef_ref[...]
  nmask = nm_ref[...]
  am2 = am_ref[...].reshape(_BK, _K)

  @pl.when(e == 0)
  def _prologue():
    hcc_ref[...] = _lncore(nf)
    hnc_ref[...] = _lncore(nbr_ref[...])
    logits = _dot(nf, wg_ref[...])
    iot = lax.broadcasted_iota(jnp.int32, (_BN, _E), 1)
    m1 = jnp.max(logits, -1, keepdims=True)
    i1 = jnp.min(jnp.where(logits == m1, iot, _E), -1, keepdims=True)
    rest = jnp.where(iot == i1, -1e30, logits)
    m2 = jnp.max(rest, -1, keepdims=True)
    i2 = jnp.min(jnp.where(rest == m2, iot, _E), -1, keepdims=True)
    e21 = jnp.exp(m2 - m1)
    p1 = 1.0 / (1.0 + e21)
    p2 = 1.0 - p1
    gat_ref[...] = (jnp.where(iot == i1, p1, 0.0) +
                    jnp.where(iot == i2, p2, 0.0))
    no_ref[...] = jnp.zeros((_BN, _H), jnp.float32)
    eo_ref[...] = jnp.zeros((_BK, _H), jnp.float32)

  hc_core = hcc_ref[...]
  hn_core = hnc_ref[...]
  gates = gat_ref[...]

  amS = _dot(am2, msel_ref[...])
  nmb = jnp.broadcast_to(nmask[:, :, None], (_BN, _K, _H)).reshape(_BK, _H)
  cnt = jnp.sum(nmask, -1, keepdims=True) + 1e-5
  onesb = onesb_ref[...]

  ga = lag_ref[0]
  ba = lab_ref[0]
  hc = hc_core * ga + ba
  hn = hn_core * ga + ba

  edge_hidden = _silu(_dot(ea, We_ref[0]) + be_ref[0])
  Wn_e = Wn_ref[0]
  cm = _dotb(hc, Wn_e[:_H])
  node_hidden = _silu(_rep8(cm) + _dotb(hn, Wn_e[_H:]) + bn_ref[0])
  Wm_e = Wm_ref[0]
  message = _silu(_dotb(edge_hidden, Wm_e[:_H]) +
                  _dotb(node_hidden, Wm_e[_H:]) + bm_ref[0])
  mb = message.astype(jnp.bfloat16)

  scale = np.float32(1.0 / np.sqrt(_DH))
  q16 = ((jnp.dot(mb, Wq_ref[0], preferred_element_type=jnp.float32) +
          bq_ref[0]) * scale).astype(jnp.bfloat16)
  kk16 = (jnp.dot(mb, Wk_ref[0], preferred_element_type=jnp.float32) +
          bk_ref[0]).astype(jnp.bfloat16)
  v = jnp.dot(mb, Wv_ref[0], preferred_element_type=jnp.float32) + bv_ref[0]
  kk3 = kk16.reshape(_BN, _K, _H)
  v3 = v.reshape(_BN, _K, _H)

  S = amS
  for k in range(_K):
    S = S + jnp.dot(q16 * _rep8(kk3[:, k, :]), selk_ref[k],
                    preferred_element_type=jnp.float32)
  Smax = jnp.max(S, -1, keepdims=True)
  Ex = jnp.exp(S - Smax)
  A16 = (Ex / _dot(Ex, onesb)).astype(jnp.bfloat16)

  o = jnp.zeros((_BK, _H), jnp.float32)
  for k in range(_K):
    o = o + jnp.dot(A16, expk_ref[k],
                    preferred_element_type=jnp.float32) * _rep8(v3[:, k, :])
  edge_out = _dotb(o, Wo_ref[0]) + bo_ref[0]

  em = (edge_out * nmb).reshape(_BN, _K, _H)
  node_sum = em[:, 0, :]
  for k in range(1, _K):
    node_sum = node_sum + em[:, k, :]
  node_out = node_sum / cnt

  node_f = node_out + nf
  edge_f = edge_out + ef
  nh = _lncore(node_f) * lng_ref[0] + lnb_ref[0]
  nh = _dotb(_silu(_dotb(nh, Wn1_ref[0]) + bn1_ref[0]),
             Wn2_ref[0]) + bn2_ref[0]
  eh = _lncore(edge_f) * leg_ref[0] + leb_ref[0]
  eh = _dotb(_silu(_dotb(eh, We1_ref[0]) + be1_ref[0]),
             We2_ref[0]) + be2_ref[0]

  iot = lax.broadcasted_iota(jnp.int32, (_BN, _E), 1)
  ge = jnp.sum(jnp.where(iot == e, gates, 0.0), -1, keepdims=True)
  geb = jnp.broadcast_to(ge, (_BN, _H))
  no_ref[...] += geb * (node_f + nh)
  eo_ref[...] += _rep8(geb) * (edge_f + eh)


def _run_main(params, nf, nbr_flat, ea_flat, ef_flat, nmask, amask):
  full = lambda a: pl.BlockSpec(a.shape, lambda i: (0,) * a.ndim)
  selk = jnp.asarray(_SELK, jnp.bfloat16)
  expk = jnp.asarray(_EXPK, jnp.bfloat16)
  msel = jnp.asarray(_MSEL)
  onesb = jnp.asarray(_ONESB)
  bf = ('We', 'Wn', 'Wm', 'Wq', 'Wk', 'Wv', 'Wo', 'Wn1', 'Wn2', 'We1', 'We2')
  pvals = [params[n].astype(jnp.bfloat16) if n in bf else
           params[n].reshape(_E, 1, -1) for n in _PNAMES]
  ea_flat = ea_flat.astype(jnp.bfloat16)

  full = lambda a: pl.BlockSpec(a.shape, lambda i, j: (0,) * a.ndim)
  def perexp(a):
    return pl.BlockSpec((1,) + a.shape[1:],
                        lambda i, j: (j,) + (0,) * (a.ndim - 1))

  in_specs = [
      pl.BlockSpec((_BN, _H), lambda i, j: (i, 0)),
      pl.BlockSpec((_BK, _H), lambda i, j: (i, 0)),
      pl.BlockSpec((_BK, _H), lambda i, j: (i, 0)),
      pl.BlockSpec((_BK, _H), lambda i, j: (i, 0)),
      pl.BlockSpec((_BN, _K), lambda i, j: (i, 0)),
      pl.BlockSpec((_BN, _K, _K), lambda i, j: (i, 0, 0)),
      full(params['w_gate']),
      full(selk), full(expk), full(msel), full(onesb),
  ] + [perexp(p) for p in pvals]

  out_specs = [
      pl.BlockSpec((_BN, _H), lambda i, j: (i, 0)),
      pl.BlockSpec((_BK, _H), lambda i, j: (i, 0)),
  ]
  out_shape = [
      jax.ShapeDtypeStruct((_N, _H), jnp.float32),
      jax.ShapeDtypeStruct((_N * _K, _H), jnp.float32),
  ]
  return pl.pallas_call(
      _tc_body,
      grid=(_NB, _E),
      in_specs=in_specs,
      out_specs=out_specs,
      out_shape=out_shape,
      scratch_shapes=[
          pltpu.VMEM((_BN, _H), jnp.float32),
          pltpu.VMEM((_BK, _H), jnp.float32),
          pltpu.VMEM((_BN, _E), jnp.float32),
      ],
  )(nf, nbr_flat, ea_flat, ef_flat, nmask, amask, params['w_gate'],
    selk, expk, msel, onesb, *pvals)


def kernel(params, node_features, edge_features, edge_attr, neighbor_mask,
           attn_mask, neighbor_list):
  nf = node_features.astype(jnp.float32)
  idxf = neighbor_list.reshape(-1).astype(jnp.int32)
  idxp = jnp.concatenate(
      [idxf, jnp.zeros((_BPAD - _N * _K,), jnp.int32)])
  nbr_raw = _sc_gather(nf, idxp.reshape(_NW, _NCH, _CHUNK))
  nbr_flat = nbr_raw[:_N * _K]
  node_out, edge_flat = _run_main(
      params, nf, nbr_flat,
      edge_attr.reshape(_N * _K, _H).astype(jnp.float32),
      edge_features.reshape(_N * _K, _H).astype(jnp.float32),
      neighbor_mask.astype(jnp.float32), attn_mask.astype(jnp.float32))
  return node_out, edge_flat.reshape(_N, _K, _H)
